# Initial kernel scaffold; baseline (speedup 1.0000x reference)
import jax
import jax.numpy as jnp
from jax.experimental import pallas as pl


def kernel(x, pe_weight):
    raise NotImplementedError("write your pallas kernel here")


# speedup vs baseline: 2.8583x; 2.8583x over previous
import jax
import jax.numpy as jnp
from jax.experimental import pallas as pl


def _add_kernel(x_ref, pe_ref, o_ref):
    o_ref[...] = x_ref[...] + pe_ref[...]


def kernel(x, pe_weight):
    B, T, D = x.shape
    BT = 512
    grid = (T // BT, B)
    return pl.pallas_call(
        _add_kernel,
        grid=grid,
        in_specs=[
            pl.BlockSpec((1, BT, D), lambda i, b: (b, i, 0)),
            pl.BlockSpec((BT, D), lambda i, b: (i, 0)),
        ],
        out_specs=pl.BlockSpec((1, BT, D), lambda i, b: (b, i, 0)),
        out_shape=jax.ShapeDtypeStruct((B, T, D), x.dtype),
    )(x, pe_weight)


# speedup vs baseline: 3.1781x; 1.1119x over previous
import jax
import jax.numpy as jnp
from jax.experimental import pallas as pl


def _add_kernel(x_ref, pe_ref, o_ref):
    o_ref[...] = x_ref[...] + pe_ref[...]


def kernel(x, pe_weight):
    B, T, D = x.shape
    BT = 1024
    grid = (T // BT, B)
    return pl.pallas_call(
        _add_kernel,
        grid=grid,
        in_specs=[
            pl.BlockSpec((1, BT, D), lambda i, b: (b, i, 0)),
            pl.BlockSpec((BT, D), lambda i, b: (i, 0)),
        ],
        out_specs=pl.BlockSpec((1, BT, D), lambda i, b: (b, i, 0)),
        out_shape=jax.ShapeDtypeStruct((B, T, D), x.dtype),
    )(x, pe_weight)


# speedup vs baseline: 3.3099x; 1.0415x over previous
import jax
import jax.numpy as jnp
from jax.experimental import pallas as pl


def _add_kernel(x_ref, pe_ref, o_ref):
    o_ref[...] = x_ref[...] + pe_ref[...]


def kernel(x, pe_weight):
    B, T, D = x.shape
    BT = 2048
    grid = (T // BT, B)
    return pl.pallas_call(
        _add_kernel,
        grid=grid,
        in_specs=[
            pl.BlockSpec((1, BT, D), lambda i, b: (b, i, 0)),
            pl.BlockSpec((BT, D), lambda i, b: (i, 0)),
        ],
        out_specs=pl.BlockSpec((1, BT, D), lambda i, b: (b, i, 0)),
        out_shape=jax.ShapeDtypeStruct((B, T, D), x.dtype),
    )(x, pe_weight)
